# baseline (speedup 1.0000x reference)
import functools

import jax
import jax.numpy as jnp
from jax import lax
from jax.experimental import pallas as pl
from jax.experimental.pallas import tpu as pltpu
from jax.experimental.pallas import tpu_sc as plsc

N_NODES = 10000
HID = 16
NC, NS = 2, 16
NW = NC * NS
CH = 80
NPAD = 10112
RPT = NPAD // NS
NB = 5
PF = 2


def _make_segsum(kc, ch, epw):
    mesh = plsc.VectorSubcoreMesh(core_axis_name="c", subcore_axis_name="s")

    @functools.partial(
        pl.kernel,
        mesh=mesh,
        compiler_params=pltpu.CompilerParams(use_tc_tiling_on_sc=False),
        out_type=jax.ShapeDtypeStruct((NC, NPAD, HID), jnp.float32),
        scratch_types=[
            pltpu.VMEM((epw,), jnp.int32),
            pltpu.VMEM((epw,), jnp.int32),
            [pltpu.VMEM((ch, HID), jnp.float32) for _ in range(NB)],
            pltpu.VMEM_SHARED((NPAD, HID), jnp.float32),
            [pltpu.SemaphoreType.DMA for _ in range(NB)],
            [pltpu.SemaphoreType.DMA for _ in range(NB)],
        ],
    )
    def segsum(z_hbm, zero_hbm, eidx_hbm, out_hbm,
               src_v, dst_v, rows, acc_sh, gsem, ssem):
        c = lax.axis_index("c")
        s = lax.axis_index("s")
        wid = c * NS + s
        nbase = s * RPT
        ebase = wid * epw
        pltpu.sync_copy(zero_hbm.at[pl.ds(nbase, RPT)],
                        acc_sh.at[pl.ds(nbase, RPT)])
        pltpu.sync_copy(eidx_hbm.at[0, pl.ds(ebase, epw)], src_v)
        pltpu.sync_copy(eidx_hbm.at[1, pl.ds(ebase, epw)], dst_v)
        plsc.subcore_barrier()

        def sidx(g):
            return src_v.at[pl.ds(g * ch, ch)]

        def didx(g):
            return dst_v.at[pl.ds(g * ch, ch)]

        for g in range(PF):
            pltpu.async_copy(z_hbm.at[sidx(g)], rows[g % NB], gsem[g % NB])

        def body(i, carry):
            for b in range(NB):
                g = i * NB + b
                bp = (b + PF) % NB

                @pl.when(g + PF >= NB)
                def _():
                    gp = g + PF - NB
                    pltpu.make_async_copy(
                        rows[bp], acc_sh.at[didx(gp)], ssem[bp]).wait()

                @pl.when(g + PF < kc)
                def _():
                    pltpu.async_copy(z_hbm.at[sidx(g + PF)], rows[bp],
                                     gsem[bp])

                pltpu.make_async_copy(z_hbm.at[sidx(g)], rows[b],
                                      gsem[b]).wait()
                pltpu.async_copy(rows[b], acc_sh.at[didx(g)], ssem[b],
                                 add=True)
            return carry

        lax.fori_loop(0, kc // NB, body, 0, unroll=False)
        for gg in range(kc - (NB - PF), kc):
            b = gg % NB
            pltpu.make_async_copy(rows[b], acc_sh.at[didx(gg)],
                                  ssem[b]).wait()
        plsc.subcore_barrier()
        pltpu.sync_copy(acc_sh.at[pl.ds(nbase, RPT)],
                        out_hbm.at[c].at[pl.ds(nbase, RPT)])

    return segsum


def _mm_body(x_ref, w_ref, o_ref):
    o_ref[...] = jnp.dot(x_ref[...], w_ref[...],
                         preferred_element_type=jnp.float32)


def _bn(v, g, b):
    m = jnp.mean(v, axis=0, keepdims=True)
    var = jnp.mean((v - m) ** 2, axis=0, keepdims=True)
    return (v - m) * lax.rsqrt(var + 1e-5) * g + b


def _layer0_body(y_ref, agg_ref, wb_ref, pk_ref, eps_ref, o_ref):
    agg = agg_ref[0, :N_NODES, :] + agg_ref[1, :N_NODES, :]
    t = jnp.maximum((1.0 + eps_ref[0, 0]) * y_ref[...] + agg
                    + pk_ref[0:1, :], 0.0)
    v = jnp.maximum(
        jnp.dot(t, wb_ref[...], preferred_element_type=jnp.float32)
        + pk_ref[1:2, :], 0.0)
    o_ref[...] = _bn(v, pk_ref[2:3, :], pk_ref[3:4, :])


def _layer_body(l, h_ref, agg_ref, wa_ref, wb_ref, pk_ref, eps_ref, o_ref):
    agg = agg_ref[0, :N_NODES, :] + agg_ref[1, :N_NODES, :]
    m = (1.0 + eps_ref[0, l]) * h_ref[...] + agg
    t = jnp.maximum(
        jnp.dot(m, wa_ref[...], preferred_element_type=jnp.float32)
        + pk_ref[4 * l:4 * l + 1, :], 0.0)
    v = jnp.maximum(
        jnp.dot(t, wb_ref[...], preferred_element_type=jnp.float32)
        + pk_ref[4 * l + 1:4 * l + 2, :], 0.0)
    o_ref[...] = _bn(v, pk_ref[4 * l + 2:4 * l + 3, :],
                     pk_ref[4 * l + 3:4 * l + 4, :])


def _layer2_head_body(h_ref, agg_ref, wa_ref, wb_ref, wl1_ref, wl2_ref,
                      bl2_ref, pk_ref, eps_ref, o_ref):
    agg = agg_ref[0, :N_NODES, :] + agg_ref[1, :N_NODES, :]
    m = (1.0 + eps_ref[0, 2]) * h_ref[...] + agg
    t = jnp.maximum(
        jnp.dot(m, wa_ref[...], preferred_element_type=jnp.float32)
        + pk_ref[8:9, :], 0.0)
    v = jnp.maximum(
        jnp.dot(t, wb_ref[...], preferred_element_type=jnp.float32)
        + pk_ref[9:10, :], 0.0)
    h = _bn(v, pk_ref[10:11, :], pk_ref[11:12, :])
    t2 = jnp.maximum(
        jnp.dot(h, wl1_ref[...], preferred_element_type=jnp.float32)
        + pk_ref[12:13, :], 0.0)
    sc = jnp.dot(t2, wl2_ref[...], preferred_element_type=jnp.float32) \
        + bl2_ref[...]
    mx = jnp.max(sc, axis=-1, keepdims=True)
    e = jnp.exp(sc - mx)
    o_ref[...] = sc - mx - jnp.log(jnp.sum(e, axis=-1, keepdims=True))


def kernel(x, edge_index, edge_weight, W0a, b0a, W0b, b0b, W1a, b1a, W1b,
           b1b, W2a, b2a, W2b, b2b, Wl1, bl1, Wl2, bl2, eps0, gamma0, beta0,
           eps1, gamma1, beta1, eps2, gamma2, beta2):
    n, f = x.shape
    e = edge_index.shape[1]
    c = Wl2.shape[1]

    eidx = edge_index.astype(jnp.int32)
    epw = e // NW
    assert e % NW == 0 and epw % CH == 0 and (epw // CH) % NB == 0
    kc = epw // CH

    zeros_pad = jnp.zeros((NPAD, HID), jnp.float32)
    packed = jnp.stack([b0a, b0b, gamma0, beta0, b1a, b1b, gamma1, beta1,
                        b2a, b2b, gamma2, beta2, bl1])
    epsv = jnp.stack([eps0, eps1, eps2]).reshape(1, 3)
    bl2r = bl2.reshape(1, c)

    mm = pl.pallas_call(
        _mm_body, out_shape=jax.ShapeDtypeStruct((n, HID), jnp.float32))
    layer0 = pl.pallas_call(
        _layer0_body, out_shape=jax.ShapeDtypeStruct((n, HID), jnp.float32))
    layer1 = pl.pallas_call(
        functools.partial(_layer_body, 1),
        out_shape=jax.ShapeDtypeStruct((n, HID), jnp.float32))
    layer2_head = pl.pallas_call(
        _layer2_head_body, out_shape=jax.ShapeDtypeStruct((n, c),
                                                          jnp.float32))
    segsum = _make_segsum(kc, CH, epw)

    y = mm(x, W0a)
    agg0 = segsum(y, zeros_pad, eidx)
    h0 = layer0(y, agg0, W0b, packed, epsv)
    agg1 = segsum(h0, zeros_pad, eidx)
    h1 = layer1(h0, agg1, W1a, W1b, packed, epsv)
    agg2 = segsum(h1, zeros_pad, eidx)
    out = layer2_head(h1, agg2, W2a, W2b, Wl1, Wl2, bl2r, packed, epsv)
    return out


# speedup vs baseline: 1.1215x; 1.1215x over previous
import functools

import jax
import jax.numpy as jnp
from jax import lax
from jax.experimental import pallas as pl
from jax.experimental.pallas import tpu as pltpu
from jax.experimental.pallas import tpu_sc as plsc

N_NODES = 10000
HID = 16
NC, NS = 2, 16
NW = NC * NS
CH = 128
NPAD = 10112
RPT = NPAD // NS
NB = 5
PF = 2


def _make_segsum(epw):
    mesh = plsc.VectorSubcoreMesh(core_axis_name="c", subcore_axis_name="s")
    full = epw // CH
    rem = epw % CH
    assert rem % 8 == 0
    nb = next(q for q in (6, 5, 4, 3, 2) if full % q == 0)
    pf = min(PF, nb - 1)

    @functools.partial(
        pl.kernel,
        mesh=mesh,
        compiler_params=pltpu.CompilerParams(use_tc_tiling_on_sc=False),
        out_type=jax.ShapeDtypeStruct((NC, NPAD, HID), jnp.float32),
        scratch_types=[
            pltpu.VMEM((epw,), jnp.int32),
            pltpu.VMEM((epw,), jnp.int32),
            [pltpu.VMEM((CH, HID), jnp.float32) for _ in range(nb)],
            pltpu.VMEM((max(rem, 8), HID), jnp.float32),
            pltpu.VMEM_SHARED((NPAD, HID), jnp.float32),
            [pltpu.SemaphoreType.DMA for _ in range(nb)],
            [pltpu.SemaphoreType.DMA for _ in range(nb)],
        ],
    )
    def segsum(z_hbm, zero_hbm, eidx_hbm, out_hbm,
               src_v, dst_v, rows, rows_t, acc_sh, gsem, ssem):
        c = lax.axis_index("c")
        s = lax.axis_index("s")
        wid = c * NS + s
        nbase = s * RPT
        ebase = wid * epw
        pltpu.sync_copy(zero_hbm.at[pl.ds(nbase, RPT)],
                        acc_sh.at[pl.ds(nbase, RPT)])
        pltpu.sync_copy(eidx_hbm.at[0, pl.ds(ebase, epw)], src_v)
        pltpu.sync_copy(eidx_hbm.at[1, pl.ds(ebase, epw)], dst_v)
        plsc.subcore_barrier()

        def sidx(g, n=CH):
            return src_v.at[pl.ds(g * CH, n)]

        def didx(g, n=CH):
            return dst_v.at[pl.ds(g * CH, n)]

        for g in range(pf):
            pltpu.async_copy(z_hbm.at[sidx(g)], rows[g % nb], gsem[g % nb])

        def body(i, carry):
            for b in range(nb):
                g = i * nb + b
                bp = (b + pf) % nb

                @pl.when(g + pf >= nb)
                def _():
                    gp = g + pf - nb
                    pltpu.make_async_copy(
                        rows[bp], acc_sh.at[didx(gp)], ssem[bp]).wait()

                @pl.when(g + pf < full)
                def _():
                    pltpu.async_copy(z_hbm.at[sidx(g + pf)], rows[bp],
                                     gsem[bp])

                pltpu.make_async_copy(z_hbm.at[sidx(g)], rows[b],
                                      gsem[b]).wait()
                pltpu.async_copy(rows[b], acc_sh.at[didx(g)], ssem[b],
                                 add=True)
            return carry

        lax.fori_loop(0, full // nb, body, 0, unroll=False)
        if rem:
            pltpu.async_copy(z_hbm.at[sidx(full, rem)], rows_t,
                             gsem[(full + pf - 1) % nb])
        for gg in range(full - (nb - pf), full):
            b = gg % nb
            pltpu.make_async_copy(rows[b], acc_sh.at[didx(gg)],
                                  ssem[b]).wait()
        if rem:
            pltpu.make_async_copy(z_hbm.at[sidx(full, rem)], rows_t,
                                  gsem[(full + pf - 1) % nb]).wait()
            pltpu.sync_copy(rows_t, acc_sh.at[didx(full, rem)], add=True)
        plsc.subcore_barrier()
        pltpu.sync_copy(acc_sh.at[pl.ds(nbase, RPT)],
                        out_hbm.at[c].at[pl.ds(nbase, RPT)])

    return segsum


def _mm_body(x_ref, w_ref, o_ref):
    o_ref[...] = jnp.dot(x_ref[...], w_ref[...],
                         preferred_element_type=jnp.float32)


def _bn(v, g, b):
    m = jnp.mean(v, axis=0, keepdims=True)
    var = jnp.mean((v - m) ** 2, axis=0, keepdims=True)
    return (v - m) * lax.rsqrt(var + 1e-5) * g + b


def _layer0_body(y_ref, agg_ref, wb_ref, pk_ref, eps_ref, o_ref):
    agg = agg_ref[0, :N_NODES, :] + agg_ref[1, :N_NODES, :]
    t = jnp.maximum((1.0 + eps_ref[0, 0]) * y_ref[...] + agg
                    + pk_ref[0:1, :], 0.0)
    v = jnp.maximum(
        jnp.dot(t, wb_ref[...], preferred_element_type=jnp.float32)
        + pk_ref[1:2, :], 0.0)
    o_ref[...] = _bn(v, pk_ref[2:3, :], pk_ref[3:4, :])


def _layer_body(l, h_ref, agg_ref, wa_ref, wb_ref, pk_ref, eps_ref, o_ref):
    agg = agg_ref[0, :N_NODES, :] + agg_ref[1, :N_NODES, :]
    m = (1.0 + eps_ref[0, l]) * h_ref[...] + agg
    t = jnp.maximum(
        jnp.dot(m, wa_ref[...], preferred_element_type=jnp.float32)
        + pk_ref[4 * l:4 * l + 1, :], 0.0)
    v = jnp.maximum(
        jnp.dot(t, wb_ref[...], preferred_element_type=jnp.float32)
        + pk_ref[4 * l + 1:4 * l + 2, :], 0.0)
    o_ref[...] = _bn(v, pk_ref[4 * l + 2:4 * l + 3, :],
                     pk_ref[4 * l + 3:4 * l + 4, :])


def _layer2_head_body(h_ref, agg_ref, wa_ref, wb_ref, wl1_ref, wl2_ref,
                      bl2_ref, pk_ref, eps_ref, o_ref):
    agg = agg_ref[0, :N_NODES, :] + agg_ref[1, :N_NODES, :]
    m = (1.0 + eps_ref[0, 2]) * h_ref[...] + agg
    t = jnp.maximum(
        jnp.dot(m, wa_ref[...], preferred_element_type=jnp.float32)
        + pk_ref[8:9, :], 0.0)
    v = jnp.maximum(
        jnp.dot(t, wb_ref[...], preferred_element_type=jnp.float32)
        + pk_ref[9:10, :], 0.0)
    h = _bn(v, pk_ref[10:11, :], pk_ref[11:12, :])
    t2 = jnp.maximum(
        jnp.dot(h, wl1_ref[...], preferred_element_type=jnp.float32)
        + pk_ref[12:13, :], 0.0)
    sc = jnp.dot(t2, wl2_ref[...], preferred_element_type=jnp.float32) \
        + bl2_ref[...]
    mx = jnp.max(sc, axis=-1, keepdims=True)
    e = jnp.exp(sc - mx)
    o_ref[...] = sc - mx - jnp.log(jnp.sum(e, axis=-1, keepdims=True))


def kernel(x, edge_index, edge_weight, W0a, b0a, W0b, b0b, W1a, b1a, W1b,
           b1b, W2a, b2a, W2b, b2b, Wl1, bl1, Wl2, bl2, eps0, gamma0, beta0,
           eps1, gamma1, beta1, eps2, gamma2, beta2):
    n, f = x.shape
    e = edge_index.shape[1]
    c = Wl2.shape[1]

    eidx = edge_index.astype(jnp.int32)
    epw = e // NW
    assert e % NW == 0 and epw % 8 == 0

    zeros_pad = jnp.zeros((NPAD, HID), jnp.float32)
    packed = jnp.stack([b0a, b0b, gamma0, beta0, b1a, b1b, gamma1, beta1,
                        b2a, b2b, gamma2, beta2, bl1])
    epsv = jnp.stack([eps0, eps1, eps2]).reshape(1, 3)
    bl2r = bl2.reshape(1, c)

    mm = pl.pallas_call(
        _mm_body, out_shape=jax.ShapeDtypeStruct((n, HID), jnp.float32))
    layer0 = pl.pallas_call(
        _layer0_body, out_shape=jax.ShapeDtypeStruct((n, HID), jnp.float32))
    layer1 = pl.pallas_call(
        functools.partial(_layer_body, 1),
        out_shape=jax.ShapeDtypeStruct((n, HID), jnp.float32))
    layer2_head = pl.pallas_call(
        _layer2_head_body, out_shape=jax.ShapeDtypeStruct((n, c),
                                                          jnp.float32))
    segsum = _make_segsum(epw)

    y = mm(x, W0a)
    agg0 = segsum(y, zeros_pad, eidx)
    h0 = layer0(y, agg0, W0b, packed, epsv)
    agg1 = segsum(h0, zeros_pad, eidx)
    h1 = layer1(h0, agg1, W1a, W1b, packed, epsv)
    agg2 = segsum(h1, zeros_pad, eidx)
    out = layer2_head(h1, agg2, W2a, W2b, Wl1, Wl2, bl2r, packed, epsv)
    return out


# speedup vs baseline: 1.2118x; 1.0805x over previous
import functools

import jax
import jax.numpy as jnp
from jax import lax
from jax.experimental import pallas as pl
from jax.experimental.pallas import tpu as pltpu
from jax.experimental.pallas import tpu_sc as plsc

N_NODES = 10000
HID = 16
NC, NS = 2, 16
NW = NC * NS
CH = 128
NPAD = 10112
RPT = NPAD // NS
NB = 5
PF = 3


def _make_segsum(epw):
    mesh = plsc.VectorSubcoreMesh(core_axis_name="c", subcore_axis_name="s")
    full = epw // CH
    rem = epw % CH
    assert rem % 8 == 0
    nb = next(q for q in (6, 5, 4, 3, 2) if full % q == 0)
    pf = min(PF, nb - 1)

    @functools.partial(
        pl.kernel,
        mesh=mesh,
        compiler_params=pltpu.CompilerParams(use_tc_tiling_on_sc=False),
        out_type=jax.ShapeDtypeStruct((NC, NPAD, HID), jnp.float32),
        scratch_types=[
            pltpu.VMEM((epw,), jnp.int32),
            pltpu.VMEM((epw,), jnp.int32),
            [pltpu.VMEM((CH, HID), jnp.float32) for _ in range(nb)],
            pltpu.VMEM((max(rem, 8), HID), jnp.float32),
            pltpu.VMEM_SHARED((NPAD, HID), jnp.float32),
            [pltpu.SemaphoreType.DMA for _ in range(nb)],
            [pltpu.SemaphoreType.DMA for _ in range(nb)],
        ],
    )
    def segsum(z_hbm, zero_hbm, eidx_hbm, out_hbm,
               src_v, dst_v, rows, rows_t, acc_sh, gsem, ssem):
        c = lax.axis_index("c")
        s = lax.axis_index("s")
        wid = c * NS + s
        nbase = s * RPT
        ebase = wid * epw
        pltpu.sync_copy(zero_hbm.at[pl.ds(nbase, RPT)],
                        acc_sh.at[pl.ds(nbase, RPT)])
        pltpu.sync_copy(eidx_hbm.at[0, pl.ds(ebase, epw)], src_v)
        pltpu.sync_copy(eidx_hbm.at[1, pl.ds(ebase, epw)], dst_v)
        plsc.subcore_barrier()

        def sidx(g, n=CH):
            return src_v.at[pl.ds(g * CH, n)]

        def didx(g, n=CH):
            return dst_v.at[pl.ds(g * CH, n)]

        for g in range(pf):
            pltpu.async_copy(z_hbm.at[sidx(g)], rows[g % nb], gsem[g % nb])

        def body(i, carry):
            for b in range(nb):
                g = i * nb + b
                bp = (b + pf) % nb

                @pl.when(g + pf >= nb)
                def _():
                    gp = g + pf - nb
                    pltpu.make_async_copy(
                        rows[bp], acc_sh.at[didx(gp)], ssem[bp]).wait()

                @pl.when(g + pf < full)
                def _():
                    pltpu.async_copy(z_hbm.at[sidx(g + pf)], rows[bp],
                                     gsem[bp])

                pltpu.make_async_copy(z_hbm.at[sidx(g)], rows[b],
                                      gsem[b]).wait()
                pltpu.async_copy(rows[b], acc_sh.at[didx(g)], ssem[b],
                                 add=True)
            return carry

        lax.fori_loop(0, full // nb, body, 0, unroll=False)
        if rem:
            pltpu.async_copy(z_hbm.at[sidx(full, rem)], rows_t,
                             gsem[(full + pf - 1) % nb])
        for gg in range(full - (nb - pf), full):
            b = gg % nb
            pltpu.make_async_copy(rows[b], acc_sh.at[didx(gg)],
                                  ssem[b]).wait()
        if rem:
            pltpu.make_async_copy(z_hbm.at[sidx(full, rem)], rows_t,
                                  gsem[(full + pf - 1) % nb]).wait()
            pltpu.sync_copy(rows_t, acc_sh.at[didx(full, rem)], add=True)
        plsc.subcore_barrier()
        pltpu.sync_copy(acc_sh.at[pl.ds(nbase, RPT)],
                        out_hbm.at[c].at[pl.ds(nbase, RPT)])

    return segsum


def _mm_body(x_ref, w_ref, o_ref):
    o_ref[...] = jnp.dot(x_ref[...], w_ref[...],
                         preferred_element_type=jnp.float32)


def _bn(v, g, b):
    m = jnp.mean(v, axis=0, keepdims=True)
    var = jnp.mean((v - m) ** 2, axis=0, keepdims=True)
    return (v - m) * lax.rsqrt(var + 1e-5) * g + b


def _layer0_body(y_ref, agg_ref, wb_ref, pk_ref, eps_ref, o_ref):
    agg = agg_ref[0, :N_NODES, :] + agg_ref[1, :N_NODES, :]
    t = jnp.maximum((1.0 + eps_ref[0, 0]) * y_ref[...] + agg
                    + pk_ref[0:1, :], 0.0)
    v = jnp.maximum(
        jnp.dot(t, wb_ref[...], preferred_element_type=jnp.float32)
        + pk_ref[1:2, :], 0.0)
    o_ref[...] = _bn(v, pk_ref[2:3, :], pk_ref[3:4, :])


def _layer_body(l, h_ref, agg_ref, wa_ref, wb_ref, pk_ref, eps_ref, o_ref):
    agg = agg_ref[0, :N_NODES, :] + agg_ref[1, :N_NODES, :]
    m = (1.0 + eps_ref[0, l]) * h_ref[...] + agg
    t = jnp.maximum(
        jnp.dot(m, wa_ref[...], preferred_element_type=jnp.float32)
        + pk_ref[4 * l:4 * l + 1, :], 0.0)
    v = jnp.maximum(
        jnp.dot(t, wb_ref[...], preferred_element_type=jnp.float32)
        + pk_ref[4 * l + 1:4 * l + 2, :], 0.0)
    o_ref[...] = _bn(v, pk_ref[4 * l + 2:4 * l + 3, :],
                     pk_ref[4 * l + 3:4 * l + 4, :])


def _layer2_head_body(h_ref, agg_ref, wa_ref, wb_ref, wl1_ref, wl2_ref,
                      bl2_ref, pk_ref, eps_ref, o_ref):
    agg = agg_ref[0, :N_NODES, :] + agg_ref[1, :N_NODES, :]
    m = (1.0 + eps_ref[0, 2]) * h_ref[...] + agg
    t = jnp.maximum(
        jnp.dot(m, wa_ref[...], preferred_element_type=jnp.float32)
        + pk_ref[8:9, :], 0.0)
    v = jnp.maximum(
        jnp.dot(t, wb_ref[...], preferred_element_type=jnp.float32)
        + pk_ref[9:10, :], 0.0)
    h = _bn(v, pk_ref[10:11, :], pk_ref[11:12, :])
    t2 = jnp.maximum(
        jnp.dot(h, wl1_ref[...], preferred_element_type=jnp.float32)
        + pk_ref[12:13, :], 0.0)
    sc = jnp.dot(t2, wl2_ref[...], preferred_element_type=jnp.float32) \
        + bl2_ref[...]
    mx = jnp.max(sc, axis=-1, keepdims=True)
    e = jnp.exp(sc - mx)
    o_ref[...] = sc - mx - jnp.log(jnp.sum(e, axis=-1, keepdims=True))


def kernel(x, edge_index, edge_weight, W0a, b0a, W0b, b0b, W1a, b1a, W1b,
           b1b, W2a, b2a, W2b, b2b, Wl1, bl1, Wl2, bl2, eps0, gamma0, beta0,
           eps1, gamma1, beta1, eps2, gamma2, beta2):
    n, f = x.shape
    e = edge_index.shape[1]
    c = Wl2.shape[1]

    eidx = edge_index.astype(jnp.int32)
    epw = e // NW
    assert e % NW == 0 and epw % 8 == 0

    zeros_pad = jnp.zeros((NPAD, HID), jnp.float32)
    packed = jnp.stack([b0a, b0b, gamma0, beta0, b1a, b1b, gamma1, beta1,
                        b2a, b2b, gamma2, beta2, bl1])
    epsv = jnp.stack([eps0, eps1, eps2]).reshape(1, 3)
    bl2r = bl2.reshape(1, c)

    mm = pl.pallas_call(
        _mm_body, out_shape=jax.ShapeDtypeStruct((n, HID), jnp.float32))
    layer0 = pl.pallas_call(
        _layer0_body, out_shape=jax.ShapeDtypeStruct((n, HID), jnp.float32))
    layer1 = pl.pallas_call(
        functools.partial(_layer_body, 1),
        out_shape=jax.ShapeDtypeStruct((n, HID), jnp.float32))
    layer2_head = pl.pallas_call(
        _layer2_head_body, out_shape=jax.ShapeDtypeStruct((n, c),
                                                          jnp.float32))
    segsum = _make_segsum(epw)

    y = mm(x, W0a)
    agg0 = segsum(y, zeros_pad, eidx)
    h0 = layer0(y, agg0, W0b, packed, epsv)
    agg1 = segsum(h0, zeros_pad, eidx)
    h1 = layer1(h0, agg1, W1a, W1b, packed, epsv)
    agg2 = segsum(h1, zeros_pad, eidx)
    out = layer2_head(h1, agg2, W2a, W2b, Wl1, Wl2, bl2r, packed, epsv)
    return out


# speedup vs baseline: 1.2643x; 1.0433x over previous
import functools

import jax
import jax.numpy as jnp
from jax import lax
from jax.experimental import pallas as pl
from jax.experimental.pallas import tpu as pltpu
from jax.experimental.pallas import tpu_sc as plsc

N_NODES = 10000
HID = 16
NC, NS = 2, 16
NW = NC * NS
CH = 128
NPAD = 10112
RPT = NPAD // NS
NB = 5
PF = 4


def _make_segsum(epw):
    mesh = plsc.VectorSubcoreMesh(core_axis_name="c", subcore_axis_name="s")
    full = epw // CH
    rem = epw % CH
    assert rem % 8 == 0
    nb = next(q for q in (6, 5, 4, 3, 2) if full % q == 0)
    pf = min(PF, nb - 1)

    @functools.partial(
        pl.kernel,
        mesh=mesh,
        compiler_params=pltpu.CompilerParams(use_tc_tiling_on_sc=False),
        out_type=jax.ShapeDtypeStruct((NC, NPAD, HID), jnp.float32),
        scratch_types=[
            pltpu.VMEM((epw,), jnp.int32),
            pltpu.VMEM((epw,), jnp.int32),
            [pltpu.VMEM((CH, HID), jnp.float32) for _ in range(nb)],
            pltpu.VMEM((max(rem, 8), HID), jnp.float32),
            pltpu.VMEM_SHARED((NPAD, HID), jnp.float32),
            [pltpu.SemaphoreType.DMA for _ in range(nb)],
            [pltpu.SemaphoreType.DMA for _ in range(nb)],
        ],
    )
    def segsum(z_hbm, zero_hbm, eidx_hbm, out_hbm,
               src_v, dst_v, rows, rows_t, acc_sh, gsem, ssem):
        c = lax.axis_index("c")
        s = lax.axis_index("s")
        wid = c * NS + s
        nbase = s * RPT
        ebase = wid * epw
        pltpu.sync_copy(zero_hbm.at[pl.ds(nbase, RPT)],
                        acc_sh.at[pl.ds(nbase, RPT)])
        pltpu.sync_copy(eidx_hbm.at[0, pl.ds(ebase, epw)], src_v)
        pltpu.sync_copy(eidx_hbm.at[1, pl.ds(ebase, epw)], dst_v)
        plsc.subcore_barrier()

        def sidx(g, n=CH):
            return src_v.at[pl.ds(g * CH, n)]

        def didx(g, n=CH):
            return dst_v.at[pl.ds(g * CH, n)]

        for g in range(pf):
            pltpu.async_copy(z_hbm.at[sidx(g)], rows[g % nb], gsem[g % nb])

        def body(i, carry):
            for b in range(nb):
                g = i * nb + b
                bp = (b + pf) % nb

                @pl.when(g + pf >= nb)
                def _():
                    gp = g + pf - nb
                    pltpu.make_async_copy(
                        rows[bp], acc_sh.at[didx(gp)], ssem[bp]).wait()

                @pl.when(g + pf < full)
                def _():
                    pltpu.async_copy(z_hbm.at[sidx(g + pf)], rows[bp],
                                     gsem[bp])

                pltpu.make_async_copy(z_hbm.at[sidx(g)], rows[b],
                                      gsem[b]).wait()
                pltpu.async_copy(rows[b], acc_sh.at[didx(g)], ssem[b],
                                 add=True)
            return carry

        lax.fori_loop(0, full // nb, body, 0, unroll=False)
        if rem:
            pltpu.async_copy(z_hbm.at[sidx(full, rem)], rows_t,
                             gsem[(full + pf - 1) % nb])
        for gg in range(full - (nb - pf), full):
            b = gg % nb
            pltpu.make_async_copy(rows[b], acc_sh.at[didx(gg)],
                                  ssem[b]).wait()
        if rem:
            pltpu.make_async_copy(z_hbm.at[sidx(full, rem)], rows_t,
                                  gsem[(full + pf - 1) % nb]).wait()
            pltpu.sync_copy(rows_t, acc_sh.at[didx(full, rem)], add=True)
        plsc.subcore_barrier()
        pltpu.sync_copy(acc_sh.at[pl.ds(nbase, RPT)],
                        out_hbm.at[c].at[pl.ds(nbase, RPT)])

    return segsum


def _mm_body(x_ref, w_ref, o_ref):
    o_ref[...] = jnp.dot(x_ref[...], w_ref[...],
                         preferred_element_type=jnp.float32)


def _bn(v, g, b):
    m = jnp.mean(v, axis=0, keepdims=True)
    var = jnp.mean((v - m) ** 2, axis=0, keepdims=True)
    return (v - m) * lax.rsqrt(var + 1e-5) * g + b


def _layer0_body(y_ref, agg_ref, wb_ref, pk_ref, eps_ref, o_ref):
    agg = agg_ref[0, :N_NODES, :] + agg_ref[1, :N_NODES, :]
    t = jnp.maximum((1.0 + eps_ref[0, 0]) * y_ref[...] + agg
                    + pk_ref[0:1, :], 0.0)
    v = jnp.maximum(
        jnp.dot(t, wb_ref[...], preferred_element_type=jnp.float32)
        + pk_ref[1:2, :], 0.0)
    o_ref[...] = _bn(v, pk_ref[2:3, :], pk_ref[3:4, :])


def _layer_body(l, h_ref, agg_ref, wa_ref, wb_ref, pk_ref, eps_ref, o_ref):
    agg = agg_ref[0, :N_NODES, :] + agg_ref[1, :N_NODES, :]
    m = (1.0 + eps_ref[0, l]) * h_ref[...] + agg
    t = jnp.maximum(
        jnp.dot(m, wa_ref[...], preferred_element_type=jnp.float32)
        + pk_ref[4 * l:4 * l + 1, :], 0.0)
    v = jnp.maximum(
        jnp.dot(t, wb_ref[...], preferred_element_type=jnp.float32)
        + pk_ref[4 * l + 1:4 * l + 2, :], 0.0)
    o_ref[...] = _bn(v, pk_ref[4 * l + 2:4 * l + 3, :],
                     pk_ref[4 * l + 3:4 * l + 4, :])


def _layer2_head_body(h_ref, agg_ref, wa_ref, wb_ref, wl1_ref, wl2_ref,
                      bl2_ref, pk_ref, eps_ref, o_ref):
    agg = agg_ref[0, :N_NODES, :] + agg_ref[1, :N_NODES, :]
    m = (1.0 + eps_ref[0, 2]) * h_ref[...] + agg
    t = jnp.maximum(
        jnp.dot(m, wa_ref[...], preferred_element_type=jnp.float32)
        + pk_ref[8:9, :], 0.0)
    v = jnp.maximum(
        jnp.dot(t, wb_ref[...], preferred_element_type=jnp.float32)
        + pk_ref[9:10, :], 0.0)
    h = _bn(v, pk_ref[10:11, :], pk_ref[11:12, :])
    t2 = jnp.maximum(
        jnp.dot(h, wl1_ref[...], preferred_element_type=jnp.float32)
        + pk_ref[12:13, :], 0.0)
    sc = jnp.dot(t2, wl2_ref[...], preferred_element_type=jnp.float32) \
        + bl2_ref[...]
    mx = jnp.max(sc, axis=-1, keepdims=True)
    e = jnp.exp(sc - mx)
    o_ref[...] = sc - mx - jnp.log(jnp.sum(e, axis=-1, keepdims=True))


def kernel(x, edge_index, edge_weight, W0a, b0a, W0b, b0b, W1a, b1a, W1b,
           b1b, W2a, b2a, W2b, b2b, Wl1, bl1, Wl2, bl2, eps0, gamma0, beta0,
           eps1, gamma1, beta1, eps2, gamma2, beta2):
    n, f = x.shape
    e = edge_index.shape[1]
    c = Wl2.shape[1]

    eidx = edge_index.astype(jnp.int32)
    epw = e // NW
    assert e % NW == 0 and epw % 8 == 0

    zeros_pad = jnp.zeros((NPAD, HID), jnp.float32)
    packed = jnp.stack([b0a, b0b, gamma0, beta0, b1a, b1b, gamma1, beta1,
                        b2a, b2b, gamma2, beta2, bl1])
    epsv = jnp.stack([eps0, eps1, eps2]).reshape(1, 3)
    bl2r = bl2.reshape(1, c)

    mm = pl.pallas_call(
        _mm_body, out_shape=jax.ShapeDtypeStruct((n, HID), jnp.float32))
    layer0 = pl.pallas_call(
        _layer0_body, out_shape=jax.ShapeDtypeStruct((n, HID), jnp.float32))
    layer1 = pl.pallas_call(
        functools.partial(_layer_body, 1),
        out_shape=jax.ShapeDtypeStruct((n, HID), jnp.float32))
    layer2_head = pl.pallas_call(
        _layer2_head_body, out_shape=jax.ShapeDtypeStruct((n, c),
                                                          jnp.float32))
    segsum = _make_segsum(epw)

    y = mm(x, W0a)
    agg0 = segsum(y, zeros_pad, eidx)
    h0 = layer0(y, agg0, W0b, packed, epsv)
    agg1 = segsum(h0, zeros_pad, eidx)
    h1 = layer1(h0, agg1, W1a, W1b, packed, epsv)
    agg2 = segsum(h1, zeros_pad, eidx)
    out = layer2_head(h1, agg2, W2a, W2b, Wl1, Wl2, bl2r, packed, epsv)
    return out


# speedup vs baseline: 1.2740x; 1.0077x over previous
import functools

import jax
import jax.numpy as jnp
from jax import lax
from jax.experimental import pallas as pl
from jax.experimental.pallas import tpu as pltpu
from jax.experimental.pallas import tpu_sc as plsc

N_NODES = 10000
HID = 16
NC, NS = 2, 16
NW = NC * NS
CH = 128
NPAD = 10112
RPT = NPAD // NS
NB = 5
PF = 5


def _make_segsum(epw):
    mesh = plsc.VectorSubcoreMesh(core_axis_name="c", subcore_axis_name="s")
    full = epw // CH
    rem = epw % CH
    assert rem % 8 == 0
    nb = next(q for q in (6, 5, 4, 3, 2) if full % q == 0)
    pf = min(PF, nb - 1)

    @functools.partial(
        pl.kernel,
        mesh=mesh,
        compiler_params=pltpu.CompilerParams(use_tc_tiling_on_sc=False),
        out_type=jax.ShapeDtypeStruct((NC, NPAD, HID), jnp.float32),
        scratch_types=[
            pltpu.VMEM((epw,), jnp.int32),
            pltpu.VMEM((epw,), jnp.int32),
            [pltpu.VMEM((CH, HID), jnp.float32) for _ in range(nb)],
            pltpu.VMEM((max(rem, 8), HID), jnp.float32),
            pltpu.VMEM_SHARED((NPAD, HID), jnp.float32),
            [pltpu.SemaphoreType.DMA for _ in range(nb)],
            [pltpu.SemaphoreType.DMA for _ in range(nb)],
        ],
    )
    def segsum(z_hbm, zero_hbm, eidx_hbm, out_hbm,
               src_v, dst_v, rows, rows_t, acc_sh, gsem, ssem):
        c = lax.axis_index("c")
        s = lax.axis_index("s")
        wid = c * NS + s
        nbase = s * RPT
        ebase = wid * epw
        pltpu.sync_copy(zero_hbm.at[pl.ds(nbase, RPT)],
                        acc_sh.at[pl.ds(nbase, RPT)])
        pltpu.sync_copy(eidx_hbm.at[0, pl.ds(ebase, epw)], src_v)
        pltpu.sync_copy(eidx_hbm.at[1, pl.ds(ebase, epw)], dst_v)
        plsc.subcore_barrier()

        def sidx(g, n=CH):
            return src_v.at[pl.ds(g * CH, n)]

        def didx(g, n=CH):
            return dst_v.at[pl.ds(g * CH, n)]

        for g in range(pf):
            pltpu.async_copy(z_hbm.at[sidx(g)], rows[g % nb], gsem[g % nb])

        def body(i, carry):
            for b in range(nb):
                g = i * nb + b
                bp = (b + pf) % nb

                @pl.when(g + pf >= nb)
                def _():
                    gp = g + pf - nb
                    pltpu.make_async_copy(
                        rows[bp], acc_sh.at[didx(gp)], ssem[bp]).wait()

                @pl.when(g + pf < full)
                def _():
                    pltpu.async_copy(z_hbm.at[sidx(g + pf)], rows[bp],
                                     gsem[bp])

                pltpu.make_async_copy(z_hbm.at[sidx(g)], rows[b],
                                      gsem[b]).wait()
                pltpu.async_copy(rows[b], acc_sh.at[didx(g)], ssem[b],
                                 add=True)
            return carry

        lax.fori_loop(0, full // nb, body, 0, unroll=False)
        if rem:
            pltpu.async_copy(z_hbm.at[sidx(full, rem)], rows_t,
                             gsem[(full + pf - 1) % nb])
        for gg in range(full - (nb - pf), full):
            b = gg % nb
            pltpu.make_async_copy(rows[b], acc_sh.at[didx(gg)],
                                  ssem[b]).wait()
        if rem:
            pltpu.make_async_copy(z_hbm.at[sidx(full, rem)], rows_t,
                                  gsem[(full + pf - 1) % nb]).wait()
            pltpu.sync_copy(rows_t, acc_sh.at[didx(full, rem)], add=True)
        plsc.subcore_barrier()
        pltpu.sync_copy(acc_sh.at[pl.ds(nbase, RPT)],
                        out_hbm.at[c].at[pl.ds(nbase, RPT)])

    return segsum


def _mm_body(x_ref, w_ref, o_ref):
    o_ref[...] = jnp.dot(x_ref[...], w_ref[...],
                         preferred_element_type=jnp.float32)


def _bn(v, g, b):
    m = jnp.mean(v, axis=0, keepdims=True)
    var = jnp.mean((v - m) ** 2, axis=0, keepdims=True)
    return (v - m) * lax.rsqrt(var + 1e-5) * g + b


def _layer0_body(y_ref, agg_ref, wb_ref, pk_ref, eps_ref, o_ref):
    agg = agg_ref[0, :N_NODES, :] + agg_ref[1, :N_NODES, :]
    t = jnp.maximum((1.0 + eps_ref[0, 0]) * y_ref[...] + agg
                    + pk_ref[0:1, :], 0.0)
    v = jnp.maximum(
        jnp.dot(t, wb_ref[...], preferred_element_type=jnp.float32)
        + pk_ref[1:2, :], 0.0)
    o_ref[...] = _bn(v, pk_ref[2:3, :], pk_ref[3:4, :])


def _layer_body(l, h_ref, agg_ref, wa_ref, wb_ref, pk_ref, eps_ref, o_ref):
    agg = agg_ref[0, :N_NODES, :] + agg_ref[1, :N_NODES, :]
    m = (1.0 + eps_ref[0, l]) * h_ref[...] + agg
    t = jnp.maximum(
        jnp.dot(m, wa_ref[...], preferred_element_type=jnp.float32)
        + pk_ref[4 * l:4 * l + 1, :], 0.0)
    v = jnp.maximum(
        jnp.dot(t, wb_ref[...], preferred_element_type=jnp.float32)
        + pk_ref[4 * l + 1:4 * l + 2, :], 0.0)
    o_ref[...] = _bn(v, pk_ref[4 * l + 2:4 * l + 3, :],
                     pk_ref[4 * l + 3:4 * l + 4, :])


def _layer2_head_body(h_ref, agg_ref, wa_ref, wb_ref, wl1_ref, wl2_ref,
                      bl2_ref, pk_ref, eps_ref, o_ref):
    agg = agg_ref[0, :N_NODES, :] + agg_ref[1, :N_NODES, :]
    m = (1.0 + eps_ref[0, 2]) * h_ref[...] + agg
    t = jnp.maximum(
        jnp.dot(m, wa_ref[...], preferred_element_type=jnp.float32)
        + pk_ref[8:9, :], 0.0)
    v = jnp.maximum(
        jnp.dot(t, wb_ref[...], preferred_element_type=jnp.float32)
        + pk_ref[9:10, :], 0.0)
    h = _bn(v, pk_ref[10:11, :], pk_ref[11:12, :])
    t2 = jnp.maximum(
        jnp.dot(h, wl1_ref[...], preferred_element_type=jnp.float32)
        + pk_ref[12:13, :], 0.0)
    sc = jnp.dot(t2, wl2_ref[...], preferred_element_type=jnp.float32) \
        + bl2_ref[...]
    mx = jnp.max(sc, axis=-1, keepdims=True)
    e = jnp.exp(sc - mx)
    o_ref[...] = sc - mx - jnp.log(jnp.sum(e, axis=-1, keepdims=True))


def kernel(x, edge_index, edge_weight, W0a, b0a, W0b, b0b, W1a, b1a, W1b,
           b1b, W2a, b2a, W2b, b2b, Wl1, bl1, Wl2, bl2, eps0, gamma0, beta0,
           eps1, gamma1, beta1, eps2, gamma2, beta2):
    n, f = x.shape
    e = edge_index.shape[1]
    c = Wl2.shape[1]

    eidx = edge_index.astype(jnp.int32)
    epw = e // NW
    assert e % NW == 0 and epw % 8 == 0

    zeros_pad = jnp.zeros((NPAD, HID), jnp.float32)
    packed = jnp.stack([b0a, b0b, gamma0, beta0, b1a, b1b, gamma1, beta1,
                        b2a, b2b, gamma2, beta2, bl1])
    epsv = jnp.stack([eps0, eps1, eps2]).reshape(1, 3)
    bl2r = bl2.reshape(1, c)

    mm = pl.pallas_call(
        _mm_body, out_shape=jax.ShapeDtypeStruct((n, HID), jnp.float32))
    layer0 = pl.pallas_call(
        _layer0_body, out_shape=jax.ShapeDtypeStruct((n, HID), jnp.float32))
    layer1 = pl.pallas_call(
        functools.partial(_layer_body, 1),
        out_shape=jax.ShapeDtypeStruct((n, HID), jnp.float32))
    layer2_head = pl.pallas_call(
        _layer2_head_body, out_shape=jax.ShapeDtypeStruct((n, c),
                                                          jnp.float32))
    segsum = _make_segsum(epw)

    y = mm(x, W0a)
    agg0 = segsum(y, zeros_pad, eidx)
    h0 = layer0(y, agg0, W0b, packed, epsv)
    agg1 = segsum(h0, zeros_pad, eidx)
    h1 = layer1(h0, agg1, W1a, W1b, packed, epsv)
    agg2 = segsum(h1, zeros_pad, eidx)
    out = layer2_head(h1, agg2, W2a, W2b, Wl1, Wl2, bl2r, packed, epsv)
    return out


# speedup vs baseline: 1.2946x; 1.0161x over previous
import functools

import jax
import jax.numpy as jnp
from jax import lax
from jax.experimental import pallas as pl
from jax.experimental.pallas import tpu as pltpu
from jax.experimental.pallas import tpu_sc as plsc

N_NODES = 10000
HID = 16
NC, NS = 2, 16
NW = NC * NS
CH = 128
NPAD = 10112
RPT = NPAD // NS
NB = 5
PF = 8


def _make_segsum(epw):
    mesh = plsc.VectorSubcoreMesh(core_axis_name="c", subcore_axis_name="s")
    full = epw // CH
    rem = epw % CH
    assert rem % 8 == 0
    nb = next(q for q in (13, 6, 5, 4, 3, 2) if full % q == 0)
    pf = min(PF, nb - 1)

    @functools.partial(
        pl.kernel,
        mesh=mesh,
        compiler_params=pltpu.CompilerParams(use_tc_tiling_on_sc=False),
        out_type=jax.ShapeDtypeStruct((NC, NPAD, HID), jnp.float32),
        scratch_types=[
            pltpu.VMEM((epw,), jnp.int32),
            pltpu.VMEM((epw,), jnp.int32),
            [pltpu.VMEM((CH, HID), jnp.float32) for _ in range(nb)],
            pltpu.VMEM((max(rem, 8), HID), jnp.float32),
            pltpu.VMEM_SHARED((NPAD, HID), jnp.float32),
            [pltpu.SemaphoreType.DMA for _ in range(nb)],
            [pltpu.SemaphoreType.DMA for _ in range(nb)],
        ],
    )
    def segsum(z_hbm, zero_hbm, eidx_hbm, out_hbm,
               src_v, dst_v, rows, rows_t, acc_sh, gsem, ssem):
        c = lax.axis_index("c")
        s = lax.axis_index("s")
        wid = c * NS + s
        nbase = s * RPT
        ebase = wid * epw
        pltpu.sync_copy(zero_hbm.at[pl.ds(nbase, RPT)],
                        acc_sh.at[pl.ds(nbase, RPT)])
        pltpu.sync_copy(eidx_hbm.at[0, pl.ds(ebase, epw)], src_v)
        pltpu.sync_copy(eidx_hbm.at[1, pl.ds(ebase, epw)], dst_v)
        plsc.subcore_barrier()

        def sidx(g, n=CH):
            return src_v.at[pl.ds(g * CH, n)]

        def didx(g, n=CH):
            return dst_v.at[pl.ds(g * CH, n)]

        for g in range(pf):
            pltpu.async_copy(z_hbm.at[sidx(g)], rows[g % nb], gsem[g % nb])

        def body(i, carry):
            for b in range(nb):
                g = i * nb + b
                bp = (b + pf) % nb

                @pl.when(g + pf >= nb)
                def _():
                    gp = g + pf - nb
                    pltpu.make_async_copy(
                        rows[bp], acc_sh.at[didx(gp)], ssem[bp]).wait()

                @pl.when(g + pf < full)
                def _():
                    pltpu.async_copy(z_hbm.at[sidx(g + pf)], rows[bp],
                                     gsem[bp])

                pltpu.make_async_copy(z_hbm.at[sidx(g)], rows[b],
                                      gsem[b]).wait()
                pltpu.async_copy(rows[b], acc_sh.at[didx(g)], ssem[b],
                                 add=True)
            return carry

        lax.fori_loop(0, full // nb, body, 0, unroll=False)
        if rem:
            pltpu.async_copy(z_hbm.at[sidx(full, rem)], rows_t,
                             gsem[(full + pf - 1) % nb])
        for gg in range(full - (nb - pf), full):
            b = gg % nb
            pltpu.make_async_copy(rows[b], acc_sh.at[didx(gg)],
                                  ssem[b]).wait()
        if rem:
            pltpu.make_async_copy(z_hbm.at[sidx(full, rem)], rows_t,
                                  gsem[(full + pf - 1) % nb]).wait()
            pltpu.sync_copy(rows_t, acc_sh.at[didx(full, rem)], add=True)
        plsc.subcore_barrier()
        pltpu.sync_copy(acc_sh.at[pl.ds(nbase, RPT)],
                        out_hbm.at[c].at[pl.ds(nbase, RPT)])

    return segsum


def _mm_body(x_ref, w_ref, o_ref):
    o_ref[...] = jnp.dot(x_ref[...], w_ref[...],
                         preferred_element_type=jnp.float32)


def _bn(v, g, b):
    m = jnp.mean(v, axis=0, keepdims=True)
    var = jnp.mean((v - m) ** 2, axis=0, keepdims=True)
    return (v - m) * lax.rsqrt(var + 1e-5) * g + b


def _layer0_body(y_ref, agg_ref, wb_ref, pk_ref, eps_ref, o_ref):
    agg = agg_ref[0, :N_NODES, :] + agg_ref[1, :N_NODES, :]
    t = jnp.maximum((1.0 + eps_ref[0, 0]) * y_ref[...] + agg
                    + pk_ref[0:1, :], 0.0)
    v = jnp.maximum(
        jnp.dot(t, wb_ref[...], preferred_element_type=jnp.float32)
        + pk_ref[1:2, :], 0.0)
    o_ref[...] = _bn(v, pk_ref[2:3, :], pk_ref[3:4, :])


def _layer_body(l, h_ref, agg_ref, wa_ref, wb_ref, pk_ref, eps_ref, o_ref):
    agg = agg_ref[0, :N_NODES, :] + agg_ref[1, :N_NODES, :]
    m = (1.0 + eps_ref[0, l]) * h_ref[...] + agg
    t = jnp.maximum(
        jnp.dot(m, wa_ref[...], preferred_element_type=jnp.float32)
        + pk_ref[4 * l:4 * l + 1, :], 0.0)
    v = jnp.maximum(
        jnp.dot(t, wb_ref[...], preferred_element_type=jnp.float32)
        + pk_ref[4 * l + 1:4 * l + 2, :], 0.0)
    o_ref[...] = _bn(v, pk_ref[4 * l + 2:4 * l + 3, :],
                     pk_ref[4 * l + 3:4 * l + 4, :])


def _layer2_head_body(h_ref, agg_ref, wa_ref, wb_ref, wl1_ref, wl2_ref,
                      bl2_ref, pk_ref, eps_ref, o_ref):
    agg = agg_ref[0, :N_NODES, :] + agg_ref[1, :N_NODES, :]
    m = (1.0 + eps_ref[0, 2]) * h_ref[...] + agg
    t = jnp.maximum(
        jnp.dot(m, wa_ref[...], preferred_element_type=jnp.float32)
        + pk_ref[8:9, :], 0.0)
    v = jnp.maximum(
        jnp.dot(t, wb_ref[...], preferred_element_type=jnp.float32)
        + pk_ref[9:10, :], 0.0)
    h = _bn(v, pk_ref[10:11, :], pk_ref[11:12, :])
    t2 = jnp.maximum(
        jnp.dot(h, wl1_ref[...], preferred_element_type=jnp.float32)
        + pk_ref[12:13, :], 0.0)
    sc = jnp.dot(t2, wl2_ref[...], preferred_element_type=jnp.float32) \
        + bl2_ref[...]
    mx = jnp.max(sc, axis=-1, keepdims=True)
    e = jnp.exp(sc - mx)
    o_ref[...] = sc - mx - jnp.log(jnp.sum(e, axis=-1, keepdims=True))


def kernel(x, edge_index, edge_weight, W0a, b0a, W0b, b0b, W1a, b1a, W1b,
           b1b, W2a, b2a, W2b, b2b, Wl1, bl1, Wl2, bl2, eps0, gamma0, beta0,
           eps1, gamma1, beta1, eps2, gamma2, beta2):
    n, f = x.shape
    e = edge_index.shape[1]
    c = Wl2.shape[1]

    eidx = edge_index.astype(jnp.int32)
    epw = e // NW
    assert e % NW == 0 and epw % 8 == 0

    zeros_pad = jnp.zeros((NPAD, HID), jnp.float32)
    packed = jnp.stack([b0a, b0b, gamma0, beta0, b1a, b1b, gamma1, beta1,
                        b2a, b2b, gamma2, beta2, bl1])
    epsv = jnp.stack([eps0, eps1, eps2]).reshape(1, 3)
    bl2r = bl2.reshape(1, c)

    mm = pl.pallas_call(
        _mm_body, out_shape=jax.ShapeDtypeStruct((n, HID), jnp.float32))
    layer0 = pl.pallas_call(
        _layer0_body, out_shape=jax.ShapeDtypeStruct((n, HID), jnp.float32))
    layer1 = pl.pallas_call(
        functools.partial(_layer_body, 1),
        out_shape=jax.ShapeDtypeStruct((n, HID), jnp.float32))
    layer2_head = pl.pallas_call(
        _layer2_head_body, out_shape=jax.ShapeDtypeStruct((n, c),
                                                          jnp.float32))
    segsum = _make_segsum(epw)

    y = mm(x, W0a)
    agg0 = segsum(y, zeros_pad, eidx)
    h0 = layer0(y, agg0, W0b, packed, epsv)
    agg1 = segsum(h0, zeros_pad, eidx)
    h1 = layer1(h0, agg1, W1a, W1b, packed, epsv)
    agg2 = segsum(h1, zeros_pad, eidx)
    out = layer2_head(h1, agg2, W2a, W2b, Wl1, Wl2, bl2r, packed, epsv)
    return out


# speedup vs baseline: 1.2949x; 1.0003x over previous
import functools

import jax
import jax.numpy as jnp
from jax import lax
from jax.experimental import pallas as pl
from jax.experimental.pallas import tpu as pltpu
from jax.experimental.pallas import tpu_sc as plsc

N_NODES = 10000
HID = 16
NC, NS = 2, 16
NW = NC * NS
CH = 128
NPAD = 10112
RPT = NPAD // NS
PF = 8


def _make_segsum(epw):
    mesh = plsc.VectorSubcoreMesh(core_axis_name="c", subcore_axis_name="s")
    full = epw // CH
    rem = epw % CH
    assert rem % 8 == 0
    nb = next(q for q in (13, 6, 5, 4, 3, 2) if full % q == 0)
    pf = min(PF, nb - 1)

    @functools.partial(
        pl.kernel,
        mesh=mesh,
        compiler_params=pltpu.CompilerParams(use_tc_tiling_on_sc=False),
        out_type=jax.ShapeDtypeStruct((NC, NPAD, HID), jnp.float32),
        scratch_types=[
            pltpu.VMEM((epw,), jnp.int32),
            pltpu.VMEM((epw,), jnp.int32),
            [pltpu.VMEM((CH, HID), jnp.float32) for _ in range(nb)],
            pltpu.VMEM((max(rem, 8), HID), jnp.float32),
            pltpu.VMEM_SHARED((NPAD, HID), jnp.float32),
            [pltpu.SemaphoreType.DMA for _ in range(nb)],
            [pltpu.SemaphoreType.DMA for _ in range(nb)],
        ],
    )
    def segsum(z_hbm, zero_hbm, eidx_hbm, out_hbm,
               src_v, dst_v, rows, rows_t, acc_sh, gsem, ssem):
        c = lax.axis_index("c")
        s = lax.axis_index("s")
        wid = c * NS + s
        nbase = s * RPT
        ebase = wid * epw
        pltpu.sync_copy(zero_hbm.at[pl.ds(nbase, RPT)],
                        acc_sh.at[pl.ds(nbase, RPT)])
        pltpu.sync_copy(eidx_hbm.at[0, pl.ds(ebase, epw)], src_v)
        pltpu.sync_copy(eidx_hbm.at[1, pl.ds(ebase, epw)], dst_v)
        plsc.subcore_barrier()

        def sidx(g, n=CH):
            return src_v.at[pl.ds(g * CH, n)]

        def didx(g, n=CH):
            return dst_v.at[pl.ds(g * CH, n)]

        for g in range(pf):
            pltpu.async_copy(z_hbm.at[sidx(g)], rows[g % nb], gsem[g % nb])

        def body(i, carry):
            for b in range(nb):
                g = i * nb + b
                bp = (b + pf) % nb

                @pl.when(g + pf >= nb)
                def _():
                    gp = g + pf - nb
                    pltpu.make_async_copy(
                        rows[bp], acc_sh.at[didx(gp)], ssem[bp]).wait()

                @pl.when(g + pf < full)
                def _():
                    pltpu.async_copy(z_hbm.at[sidx(g + pf)], rows[bp],
                                     gsem[bp])

                pltpu.make_async_copy(z_hbm.at[sidx(g)], rows[b],
                                      gsem[b]).wait()
                pltpu.async_copy(rows[b], acc_sh.at[didx(g)], ssem[b],
                                 add=True)
            return carry

        lax.fori_loop(0, full // nb, body, 0, unroll=False)
        if rem:
            pltpu.async_copy(z_hbm.at[sidx(full, rem)], rows_t,
                             gsem[(full + pf - 1) % nb])
        for gg in range(full - (nb - pf), full):
            b = gg % nb
            pltpu.make_async_copy(rows[b], acc_sh.at[didx(gg)],
                                  ssem[b]).wait()
        if rem:
            pltpu.make_async_copy(z_hbm.at[sidx(full, rem)], rows_t,
                                  gsem[(full + pf - 1) % nb]).wait()
            pltpu.sync_copy(rows_t, acc_sh.at[didx(full, rem)], add=True)
        plsc.subcore_barrier()
        pltpu.sync_copy(acc_sh.at[pl.ds(nbase, RPT)],
                        out_hbm.at[c].at[pl.ds(nbase, RPT)])

    return segsum


def _mm_body(x_ref, w_ref, o_ref):
    o_ref[...] = jnp.dot(x_ref[...], w_ref[...],
                         preferred_element_type=jnp.float32)


def _bn(v, g, b):
    m = jnp.mean(v, axis=0, keepdims=True)
    var = jnp.mean((v - m) ** 2, axis=0, keepdims=True)
    return (v - m) * lax.rsqrt(var + 1e-5) * g + b


def _layer0_body(y_ref, agg_ref, wb_ref, pk_ref, eps_ref, o_ref):
    agg = agg_ref[0, :N_NODES, :] + agg_ref[1, :N_NODES, :]
    t = jnp.maximum((1.0 + eps_ref[0, 0]) * y_ref[...] + agg
                    + pk_ref[0:1, :], 0.0)
    v = jnp.maximum(
        jnp.dot(t, wb_ref[...], preferred_element_type=jnp.float32)
        + pk_ref[1:2, :], 0.0)
    o_ref[...] = _bn(v, pk_ref[2:3, :], pk_ref[3:4, :])


def _layer_body(l, h_ref, agg_ref, wa_ref, wb_ref, pk_ref, eps_ref, o_ref):
    agg = agg_ref[0, :N_NODES, :] + agg_ref[1, :N_NODES, :]
    m = (1.0 + eps_ref[0, l]) * h_ref[...] + agg
    t = jnp.maximum(
        jnp.dot(m, wa_ref[...], preferred_element_type=jnp.float32)
        + pk_ref[4 * l:4 * l + 1, :], 0.0)
    v = jnp.maximum(
        jnp.dot(t, wb_ref[...], preferred_element_type=jnp.float32)
        + pk_ref[4 * l + 1:4 * l + 2, :], 0.0)
    o_ref[...] = _bn(v, pk_ref[4 * l + 2:4 * l + 3, :],
                     pk_ref[4 * l + 3:4 * l + 4, :])


def _layer2_head_body(h_ref, agg_ref, wa_ref, wb_ref, wl1_ref, wl2_ref,
                      bl2_ref, pk_ref, eps_ref, o_ref):
    agg = agg_ref[0, :N_NODES, :] + agg_ref[1, :N_NODES, :]
    m = (1.0 + eps_ref[0, 2]) * h_ref[...] + agg
    t = jnp.maximum(
        jnp.dot(m, wa_ref[...], preferred_element_type=jnp.float32)
        + pk_ref[8:9, :], 0.0)
    v = jnp.maximum(
        jnp.dot(t, wb_ref[...], preferred_element_type=jnp.float32)
        + pk_ref[9:10, :], 0.0)
    h = _bn(v, pk_ref[10:11, :], pk_ref[11:12, :])
    t2 = jnp.maximum(
        jnp.dot(h, wl1_ref[...], preferred_element_type=jnp.float32)
        + pk_ref[12:13, :], 0.0)
    sc = jnp.dot(t2, wl2_ref[...], preferred_element_type=jnp.float32) \
        + bl2_ref[...]
    mx = jnp.max(sc, axis=-1, keepdims=True)
    e = jnp.exp(sc - mx)
    o_ref[...] = sc - mx - jnp.log(jnp.sum(e, axis=-1, keepdims=True))


def kernel(x, edge_index, edge_weight, W0a, b0a, W0b, b0b, W1a, b1a, W1b,
           b1b, W2a, b2a, W2b, b2b, Wl1, bl1, Wl2, bl2, eps0, gamma0, beta0,
           eps1, gamma1, beta1, eps2, gamma2, beta2):
    n, f = x.shape
    e = edge_index.shape[1]
    c = Wl2.shape[1]

    eidx = edge_index.astype(jnp.int32)
    epw = e // NW
    assert e % NW == 0 and epw % 8 == 0

    zeros_pad = jnp.zeros((NPAD, HID), jnp.float32)
    packed = jnp.stack([b0a, b0b, gamma0, beta0, b1a, b1b, gamma1, beta1,
                        b2a, b2b, gamma2, beta2, bl1])
    epsv = jnp.stack([eps0, eps1, eps2]).reshape(1, 3)
    bl2r = bl2.reshape(1, c)

    mm = pl.pallas_call(
        _mm_body, out_shape=jax.ShapeDtypeStruct((n, HID), jnp.float32))
    layer0 = pl.pallas_call(
        _layer0_body, out_shape=jax.ShapeDtypeStruct((n, HID), jnp.float32))
    layer1 = pl.pallas_call(
        functools.partial(_layer_body, 1),
        out_shape=jax.ShapeDtypeStruct((n, HID), jnp.float32))
    layer2_head = pl.pallas_call(
        _layer2_head_body, out_shape=jax.ShapeDtypeStruct((n, c),
                                                          jnp.float32))
    segsum = _make_segsum(epw)

    y = mm(x, W0a)
    agg0 = segsum(y, zeros_pad, eidx)
    h0 = layer0(y, agg0, W0b, packed, epsv)
    agg1 = segsum(h0, zeros_pad, eidx)
    h1 = layer1(h0, agg1, W1a, W1b, packed, epsv)
    agg2 = segsum(h1, zeros_pad, eidx)
    out = layer2_head(h1, agg2, W2a, W2b, Wl1, Wl2, bl2r, packed, epsv)
    return out
